# Initial kernel scaffold; baseline (speedup 1.0000x reference)
import jax
import jax.numpy as jnp
from jax.experimental import pallas as pl


def kernel(pcd, feat, pcd_feadb, feat_feadb, pos_w1, pos_b1, pos_g1, pos_be1, pos_w2, pos_b2, attn_w1, attn_b1, attn_g1, attn_be1, attn_w2, attn_b2):
    raise NotImplementedError("write your pallas kernel here")


# speedup vs baseline: 10.3939x; 10.3939x over previous
import functools

import jax
import jax.numpy as jnp
from jax.experimental import pallas as pl
from jax.experimental.pallas import tpu as pltpu
from jax.experimental.pallas import tpu_sc as plsc

N_NEI = 16
D_TAB = 80
EPS = 1e-5


def _knn_body(q_ref, r_ref, idx_ref, *, m_total):
    b = pl.program_id(0)
    q = q_ref[0]
    r = r_ref[0]
    qs = jnp.sum(q * q, axis=1, keepdims=True)
    rs = jnp.sum(r * r, axis=1)[None, :]
    d = qs + rs - 2.0 * jax.lax.dot_general(
        q, r, (((1,), (1,)), ((), ())), preferred_element_type=jnp.float32)
    iota = jax.lax.broadcasted_iota(jnp.int32, d.shape, 1)
    big = jnp.int32(m_total)
    cols = []
    for _ in range(N_NEI):
        mv = jnp.min(d, axis=1, keepdims=True)
        ji = jnp.min(jnp.where(d == mv, iota, big), axis=1, keepdims=True)
        cols.append(ji)
        d = jnp.where(iota == ji, jnp.float32(jnp.inf), d)
    idx_ref[0] = jnp.concatenate(cols, axis=1) + b * m_total


def _knn(qT, rT):
    B, N, C = qT.shape
    M = rT.shape[1]
    NQ = 256
    return pl.pallas_call(
        functools.partial(_knn_body, m_total=M),
        grid=(B, N // NQ),
        in_specs=[
            pl.BlockSpec((1, NQ, C), lambda b, i: (b, i, 0)),
            pl.BlockSpec((1, M, C), lambda b, i: (b, 0, 0)),
        ],
        out_specs=pl.BlockSpec((1, NQ, N_NEI), lambda b, i: (b, i, 0)),
        out_shape=jax.ShapeDtypeStruct((B, N, N_NEI), jnp.int32),
    )(qT, rT)


def _sc_gather(table, idx_flat):
    rows_total = idx_flat.shape[0]
    d = table.shape[1]
    info = plsc.get_sparse_core_info()
    nw = info.num_cores * info.num_subcores
    per_w = rows_total // nw
    ch = 128
    n_ch = per_w // ch
    mesh = plsc.VectorSubcoreMesh(core_axis_name="c", subcore_axis_name="s")

    @functools.partial(
        pl.kernel,
        out_type=jax.ShapeDtypeStruct((rows_total, d), jnp.float32),
        mesh=mesh,
        scratch_types=[
            pltpu.VMEM((ch,), jnp.int32),
            pltpu.VMEM((ch, d), jnp.float32),
            pltpu.SemaphoreType.DMA,
        ],
        compiler_params=pltpu.CompilerParams(use_tc_tiling_on_sc=False),
    )
    def k(table_hbm, idx_hbm, out_hbm, idx_v, rows_v, sem):
        wid = jax.lax.axis_index("s") * info.num_cores + jax.lax.axis_index("c")
        base = wid * per_w

        def body(c, carry):
            off = base + c * ch
            pltpu.sync_copy(idx_hbm.at[pl.ds(off, ch)], idx_v)
            pltpu.async_copy(table_hbm.at[idx_v], rows_v, sem).wait()
            pltpu.sync_copy(rows_v, out_hbm.at[pl.ds(off, ch)])
            return carry

        jax.lax.fori_loop(0, n_ch, body, 0)

    return k(table, idx_flat)


def _stats1_body(gp_ref, p_ref, acc_ref):
    pb = p_ref.shape[0]
    gp = gp_ref[:, 64:80]
    p = p_ref[...]
    prep = jnp.reshape(
        jnp.broadcast_to(p[:, None, :], (pb, N_NEI, 16)), (pb * N_NEI, 16))
    pr = prep - gp
    outer = jax.lax.dot_general(
        pr, pr, (((0,), (0,)), ((), ())), preferred_element_type=jnp.float32)
    s = jnp.sum(pr, axis=0)

    @pl.when(pl.program_id(0) == 0)
    def _():
        acc_ref[...] = jnp.zeros_like(acc_ref)

    acc_ref[0:16, :] += outer
    acc_ref[16:17, :] += s[None, :]


def _stats1(g, pcd16, rb):
    rows = g.shape[0]
    pb = rb // N_NEI
    return pl.pallas_call(
        _stats1_body,
        grid=(rows // rb,),
        in_specs=[
            pl.BlockSpec((rb, D_TAB), lambda i: (i, 0)),
            pl.BlockSpec((pb, 16), lambda i: (i, 0)),
        ],
        out_specs=pl.BlockSpec((24, 16), lambda i: (0, 0)),
        out_shape=jax.ShapeDtypeStruct((24, 16), jnp.float32),
    )(g, pcd16)


def _pe_body(g_ref, p_ref, f_ref, acc1_ref, w1_ref, b1_ref, g1_ref, be1_ref,
             w2_ref, b2_ref, pe_ref, acc2_ref, *, cnt):
    pb = p_ref.shape[0]
    rb = pb * N_NEI
    s = acc1_ref[16:17, :]
    outer = acc1_ref[0:16, :]
    mean_p = s / cnt
    cov = outer / cnt - mean_p * jnp.reshape(mean_p, (16, 1))
    w1 = w1_ref[...]
    mean1 = jax.lax.dot_general(
        mean_p, w1, (((1,), (1,)), ((), ())),
        preferred_element_type=jnp.float32) + b1_ref[...]
    wc = jax.lax.dot_general(
        w1, cov, (((1,), (0,)), ((), ())), preferred_element_type=jnp.float32)
    var1 = jnp.reshape(jnp.sum(wc * w1, axis=1), (1, 64))

    gp = g_ref[:, 64:80]
    p = p_ref[...]
    prep = jnp.reshape(
        jnp.broadcast_to(p[:, None, :], (pb, N_NEI, 16)), (rb, 16))
    pr = prep - gp
    pe1 = jax.lax.dot_general(
        pr, w1, (((1,), (1,)), ((), ())),
        preferred_element_type=jnp.float32) + b1_ref[...]
    xn = (pe1 - mean1) * jax.lax.rsqrt(var1 + EPS) * g1_ref[...] + be1_ref[...]
    z = jnp.maximum(xn, 0.0)
    pe = jax.lax.dot_general(
        z, w2_ref[...], (((1,), (1,)), ((), ())),
        preferred_element_type=jnp.float32) + b2_ref[...]
    pe_ref[...] = pe

    f = f_ref[...]
    frep = jnp.reshape(
        jnp.broadcast_to(f[:, None, :], (pb, N_NEI, 64)), (rb, 64))
    x2 = (frep - g_ref[:, 0:64]) + pe
    outer2 = jax.lax.dot_general(
        x2, x2, (((0,), (0,)), ((), ())), preferred_element_type=jnp.float32)
    s2 = jnp.sum(x2, axis=0)

    @pl.when(pl.program_id(0) == 0)
    def _():
        acc2_ref[...] = jnp.zeros_like(acc2_ref)

    acc2_ref[0:64, :] += outer2
    acc2_ref[64:65, :] += s2[None, :]


def _pe_pass(g, pcd16, featT, acc1, w1p, b1, g1, be1, w2, b2, rb):
    rows = g.shape[0]
    pb = rb // N_NEI
    cnt = float(rows)
    return pl.pallas_call(
        functools.partial(_pe_body, cnt=cnt),
        grid=(rows // rb,),
        in_specs=[
            pl.BlockSpec((rb, D_TAB), lambda i: (i, 0)),
            pl.BlockSpec((pb, 16), lambda i: (i, 0)),
            pl.BlockSpec((pb, 64), lambda i: (i, 0)),
            pl.BlockSpec((24, 16), lambda i: (0, 0)),
            pl.BlockSpec((64, 16), lambda i: (0, 0)),
            pl.BlockSpec((1, 64), lambda i: (0, 0)),
            pl.BlockSpec((1, 64), lambda i: (0, 0)),
            pl.BlockSpec((1, 64), lambda i: (0, 0)),
            pl.BlockSpec((64, 64), lambda i: (0, 0)),
            pl.BlockSpec((1, 64), lambda i: (0, 0)),
        ],
        out_specs=[
            pl.BlockSpec((rb, 64), lambda i: (i, 0)),
            pl.BlockSpec((72, 64), lambda i: (0, 0)),
        ],
        out_shape=[
            jax.ShapeDtypeStruct((rows, 64), jnp.float32),
            jax.ShapeDtypeStruct((72, 64), jnp.float32),
        ],
    )(g, pcd16, featT, acc1, w1p, b1, g1, be1, w2, b2)


def _final_body(g_ref, pe_ref, f_ref, acc2_ref, w1_ref, b1_ref, g1_ref,
                be1_ref, w2_ref, b2_ref, out_ref, *, cnt):
    pb = f_ref.shape[0]
    rb = pb * N_NEI
    hid = w1_ref.shape[0]
    s2 = acc2_ref[64:65, :]
    outer2 = acc2_ref[0:64, :]
    mean_x = s2 / cnt
    cov = outer2 / cnt - mean_x * jnp.reshape(mean_x, (64, 1))
    w1 = w1_ref[...]
    mean2 = jax.lax.dot_general(
        mean_x, w1, (((1,), (1,)), ((), ())),
        preferred_element_type=jnp.float32) + b1_ref[...]
    wc = jax.lax.dot_general(
        w1, cov, (((1,), (0,)), ((), ())), preferred_element_type=jnp.float32)
    var2 = jnp.reshape(jnp.sum(wc * w1, axis=1), (1, hid))

    pe = pe_ref[...]
    f = f_ref[...]
    frep = jnp.reshape(
        jnp.broadcast_to(f[:, None, :], (pb, N_NEI, 64)), (rb, 64))
    gfeat = g_ref[:, 0:64]
    x2 = (frep - gfeat) + pe
    ap = jax.lax.dot_general(
        x2, w1, (((1,), (1,)), ((), ())),
        preferred_element_type=jnp.float32) + b1_ref[...]
    an = (ap - mean2) * jax.lax.rsqrt(var2 + EPS) * g1_ref[...] + be1_ref[...]
    an = jnp.maximum(an, 0.0)
    wp = jax.lax.dot_general(
        an, w2_ref[...], (((1,), (1,)), ((), ())),
        preferred_element_type=jnp.float32) + b2_ref[...]
    wp3 = jnp.reshape(wp, (pb, N_NEI, 64))
    m = jnp.max(wp3, axis=1, keepdims=True)
    e = jnp.exp(wp3 - m)
    sm = e / jnp.sum(e, axis=1, keepdims=True)
    gf3 = jnp.reshape(gfeat + pe, (pb, N_NEI, 64))
    out_ref[...] = jnp.sum(sm * gf3, axis=1)


def _final_pass(g, pe, featT, acc2, aw1, ab1, ag1, abe1, aw2, ab2, rb):
    rows = g.shape[0]
    pb = rb // N_NEI
    hid = aw1.shape[0]
    cnt = float(rows)
    return pl.pallas_call(
        functools.partial(_final_body, cnt=cnt),
        grid=(rows // rb,),
        in_specs=[
            pl.BlockSpec((rb, D_TAB), lambda i: (i, 0)),
            pl.BlockSpec((rb, 64), lambda i: (i, 0)),
            pl.BlockSpec((pb, 64), lambda i: (i, 0)),
            pl.BlockSpec((72, 64), lambda i: (0, 0)),
            pl.BlockSpec((hid, 64), lambda i: (0, 0)),
            pl.BlockSpec((1, hid), lambda i: (0, 0)),
            pl.BlockSpec((1, hid), lambda i: (0, 0)),
            pl.BlockSpec((1, hid), lambda i: (0, 0)),
            pl.BlockSpec((64, hid), lambda i: (0, 0)),
            pl.BlockSpec((1, 64), lambda i: (0, 0)),
        ],
        out_specs=pl.BlockSpec((pb, 64), lambda i: (i, 0)),
        out_shape=jax.ShapeDtypeStruct((rows // N_NEI, 64), jnp.float32),
    )(g, pe, featT, acc2, aw1, ab1, ag1, abe1, aw2, ab2)


def kernel(pcd, feat, pcd_feadb, feat_feadb,
           pos_w1, pos_b1, pos_g1, pos_be1, pos_w2, pos_b2,
           attn_w1, attn_b1, attn_g1, attn_be1, attn_w2, attn_b2):
    B, C, N = feat.shape
    M = N + feat_feadb.shape[2]
    rows = B * N * N_NEI
    RB = 2048

    fusion_feat = jnp.concatenate([feat, feat_feadb], axis=2)
    fusion_pcd = jnp.concatenate([pcd, pcd_feadb], axis=2)
    rT = fusion_feat.transpose(0, 2, 1)
    pT = fusion_pcd.transpose(0, 2, 1)
    table = jnp.concatenate(
        [rT, pT, jnp.zeros((B, M, D_TAB - C - 3), jnp.float32)],
        axis=2).reshape(B * M, D_TAB)
    qT = feat.transpose(0, 2, 1)
    featT = qT.reshape(B * N, C)
    pcd16 = jnp.concatenate(
        [pcd.transpose(0, 2, 1).reshape(B * N, 3),
         jnp.zeros((B * N, 13), jnp.float32)], axis=1)

    idx = _knn(qT, rT)
    g = _sc_gather(table, idx.reshape(rows))
    acc1 = _stats1(g, pcd16, RB)

    w1p = jnp.concatenate(
        [pos_w1, jnp.zeros((pos_w1.shape[0], 13), jnp.float32)], axis=1)
    pe, acc2 = _pe_pass(g, pcd16, featT, acc1, w1p,
                        pos_b1[None, :], pos_g1[None, :], pos_be1[None, :],
                        pos_w2, pos_b2[None, :], RB)
    refined = _final_pass(g, pe, featT, acc2, attn_w1,
                          attn_b1[None, :], attn_g1[None, :],
                          attn_be1[None, :], attn_w2, attn_b2[None, :], RB)
    return refined.reshape(B, N, C).transpose(0, 2, 1)


# speedup vs baseline: 11.3370x; 1.0907x over previous
import functools

import jax
import jax.numpy as jnp
from jax.experimental import pallas as pl
from jax.experimental.pallas import tpu as pltpu
from jax.experimental.pallas import tpu_sc as plsc

N_NEI = 16
D_TAB = 80
EPS = 1e-5


def _knn_body(q_ref, r_ref, idx_ref, *, m_total):
    b = pl.program_id(0)
    q = q_ref[0]
    r = r_ref[0]
    qs = jnp.sum(q * q, axis=1, keepdims=True)
    rs = jnp.sum(r * r, axis=1)[None, :]
    d = qs + rs - 2.0 * jax.lax.dot_general(
        q, r, (((1,), (1,)), ((), ())), preferred_element_type=jnp.float32)
    iota = jax.lax.broadcasted_iota(jnp.int32, d.shape, 1)
    cols = []
    for _ in range(N_NEI):
        ji = jnp.argmin(d, axis=1)[:, None]
        cols.append(ji)
        d = jnp.where(iota == ji, jnp.float32(jnp.inf), d)
    idx_ref[0] = jnp.concatenate(cols, axis=1) + b * m_total


def _knn(qT, rT):
    B, N, C = qT.shape
    M = rT.shape[1]
    NQ = 256
    return pl.pallas_call(
        functools.partial(_knn_body, m_total=M),
        grid=(B, N // NQ),
        in_specs=[
            pl.BlockSpec((1, NQ, C), lambda b, i: (b, i, 0)),
            pl.BlockSpec((1, M, C), lambda b, i: (b, 0, 0)),
        ],
        out_specs=pl.BlockSpec((1, NQ, N_NEI), lambda b, i: (b, i, 0)),
        out_shape=jax.ShapeDtypeStruct((B, N, N_NEI), jnp.int32),
    )(qT, rT)


def _sc_gather(table, idx_flat):
    rows_total = idx_flat.shape[0]
    d = table.shape[1]
    info = plsc.get_sparse_core_info()
    nw = info.num_cores * info.num_subcores
    per_w = rows_total // nw
    ch = 128
    n_ch = per_w // ch
    mesh = plsc.VectorSubcoreMesh(core_axis_name="c", subcore_axis_name="s")

    @functools.partial(
        pl.kernel,
        out_type=jax.ShapeDtypeStruct((rows_total, d), jnp.float32),
        mesh=mesh,
        scratch_types=[
            pltpu.VMEM((ch,), jnp.int32),
            pltpu.VMEM((ch, d), jnp.float32),
            pltpu.SemaphoreType.DMA,
        ],
        compiler_params=pltpu.CompilerParams(use_tc_tiling_on_sc=False),
    )
    def k(table_hbm, idx_hbm, out_hbm, idx_v, rows_v, sem):
        wid = jax.lax.axis_index("s") * info.num_cores + jax.lax.axis_index("c")
        base = wid * per_w

        def body(c, carry):
            off = base + c * ch
            pltpu.sync_copy(idx_hbm.at[pl.ds(off, ch)], idx_v)
            pltpu.async_copy(table_hbm.at[idx_v], rows_v, sem).wait()
            pltpu.sync_copy(rows_v, out_hbm.at[pl.ds(off, ch)])
            return carry

        jax.lax.fori_loop(0, n_ch, body, 0)

    return k(table, idx_flat)


def _stats1_body(gp_ref, p_ref, acc_ref):
    pb = p_ref.shape[0]
    gp = gp_ref[:, 64:80]
    p = p_ref[...]
    prep = jnp.reshape(
        jnp.broadcast_to(p[:, None, :], (pb, N_NEI, 16)), (pb * N_NEI, 16))
    pr = prep - gp
    outer = jax.lax.dot_general(
        pr, pr, (((0,), (0,)), ((), ())), preferred_element_type=jnp.float32)
    s = jnp.sum(pr, axis=0)

    @pl.when(pl.program_id(0) == 0)
    def _():
        acc_ref[...] = jnp.zeros_like(acc_ref)

    acc_ref[0:16, :] += outer
    acc_ref[16:17, :] += s[None, :]


def _stats1(g, pcd16, rb):
    rows = g.shape[0]
    pb = rb // N_NEI
    return pl.pallas_call(
        _stats1_body,
        grid=(rows // rb,),
        in_specs=[
            pl.BlockSpec((rb, D_TAB), lambda i: (i, 0)),
            pl.BlockSpec((pb, 16), lambda i: (i, 0)),
        ],
        out_specs=pl.BlockSpec((24, 16), lambda i: (0, 0)),
        out_shape=jax.ShapeDtypeStruct((24, 16), jnp.float32),
    )(g, pcd16)


def _pe_body(g_ref, p_ref, f_ref, acc1_ref, w1_ref, b1_ref, g1_ref, be1_ref,
             w2_ref, b2_ref, pe_ref, acc2_ref, *, cnt):
    pb = p_ref.shape[0]
    rb = pb * N_NEI
    s = acc1_ref[16:17, :]
    outer = acc1_ref[0:16, :]
    mean_p = s / cnt
    cov = outer / cnt - mean_p * jnp.reshape(mean_p, (16, 1))
    w1 = w1_ref[...]
    mean1 = jax.lax.dot_general(
        mean_p, w1, (((1,), (1,)), ((), ())),
        preferred_element_type=jnp.float32) + b1_ref[...]
    wc = jax.lax.dot_general(
        w1, cov, (((1,), (0,)), ((), ())), preferred_element_type=jnp.float32)
    var1 = jnp.reshape(jnp.sum(wc * w1, axis=1), (1, 64))

    gp = g_ref[:, 64:80]
    p = p_ref[...]
    prep = jnp.reshape(
        jnp.broadcast_to(p[:, None, :], (pb, N_NEI, 16)), (rb, 16))
    pr = prep - gp
    pe1 = jax.lax.dot_general(
        pr, w1, (((1,), (1,)), ((), ())),
        preferred_element_type=jnp.float32) + b1_ref[...]
    xn = (pe1 - mean1) * jax.lax.rsqrt(var1 + EPS) * g1_ref[...] + be1_ref[...]
    z = jnp.maximum(xn, 0.0)
    pe = jax.lax.dot_general(
        z, w2_ref[...], (((1,), (1,)), ((), ())),
        preferred_element_type=jnp.float32) + b2_ref[...]
    pe_ref[...] = pe

    f = f_ref[...]
    frep = jnp.reshape(
        jnp.broadcast_to(f[:, None, :], (pb, N_NEI, 64)), (rb, 64))
    x2 = (frep - g_ref[:, 0:64]) + pe
    outer2 = jax.lax.dot_general(
        x2, x2, (((0,), (0,)), ((), ())), preferred_element_type=jnp.float32)
    s2 = jnp.sum(x2, axis=0)

    @pl.when(pl.program_id(0) == 0)
    def _():
        acc2_ref[...] = jnp.zeros_like(acc2_ref)

    acc2_ref[0:64, :] += outer2
    acc2_ref[64:65, :] += s2[None, :]


def _pe_pass(g, pcd16, featT, acc1, w1p, b1, g1, be1, w2, b2, rb):
    rows = g.shape[0]
    pb = rb // N_NEI
    cnt = float(rows)
    return pl.pallas_call(
        functools.partial(_pe_body, cnt=cnt),
        grid=(rows // rb,),
        in_specs=[
            pl.BlockSpec((rb, D_TAB), lambda i: (i, 0)),
            pl.BlockSpec((pb, 16), lambda i: (i, 0)),
            pl.BlockSpec((pb, 64), lambda i: (i, 0)),
            pl.BlockSpec((24, 16), lambda i: (0, 0)),
            pl.BlockSpec((64, 16), lambda i: (0, 0)),
            pl.BlockSpec((1, 64), lambda i: (0, 0)),
            pl.BlockSpec((1, 64), lambda i: (0, 0)),
            pl.BlockSpec((1, 64), lambda i: (0, 0)),
            pl.BlockSpec((64, 64), lambda i: (0, 0)),
            pl.BlockSpec((1, 64), lambda i: (0, 0)),
        ],
        out_specs=[
            pl.BlockSpec((rb, 64), lambda i: (i, 0)),
            pl.BlockSpec((72, 64), lambda i: (0, 0)),
        ],
        out_shape=[
            jax.ShapeDtypeStruct((rows, 64), jnp.float32),
            jax.ShapeDtypeStruct((72, 64), jnp.float32),
        ],
    )(g, pcd16, featT, acc1, w1p, b1, g1, be1, w2, b2)


def _final_body(g_ref, pe_ref, f_ref, acc2_ref, w1_ref, b1_ref, g1_ref,
                be1_ref, w2_ref, b2_ref, out_ref, *, cnt):
    pb = f_ref.shape[0]
    rb = pb * N_NEI
    hid = w1_ref.shape[0]
    s2 = acc2_ref[64:65, :]
    outer2 = acc2_ref[0:64, :]
    mean_x = s2 / cnt
    cov = outer2 / cnt - mean_x * jnp.reshape(mean_x, (64, 1))
    w1 = w1_ref[...]
    mean2 = jax.lax.dot_general(
        mean_x, w1, (((1,), (1,)), ((), ())),
        preferred_element_type=jnp.float32) + b1_ref[...]
    wc = jax.lax.dot_general(
        w1, cov, (((1,), (0,)), ((), ())), preferred_element_type=jnp.float32)
    var2 = jnp.reshape(jnp.sum(wc * w1, axis=1), (1, hid))

    pe = pe_ref[...]
    f = f_ref[...]
    frep = jnp.reshape(
        jnp.broadcast_to(f[:, None, :], (pb, N_NEI, 64)), (rb, 64))
    gfeat = g_ref[:, 0:64]
    x2 = (frep - gfeat) + pe
    ap = jax.lax.dot_general(
        x2, w1, (((1,), (1,)), ((), ())),
        preferred_element_type=jnp.float32) + b1_ref[...]
    an = (ap - mean2) * jax.lax.rsqrt(var2 + EPS) * g1_ref[...] + be1_ref[...]
    an = jnp.maximum(an, 0.0)
    wp = jax.lax.dot_general(
        an, w2_ref[...], (((1,), (1,)), ((), ())),
        preferred_element_type=jnp.float32) + b2_ref[...]
    wp3 = jnp.reshape(wp, (pb, N_NEI, 64))
    m = jnp.max(wp3, axis=1, keepdims=True)
    e = jnp.exp(wp3 - m)
    sm = e / jnp.sum(e, axis=1, keepdims=True)
    gf3 = jnp.reshape(gfeat + pe, (pb, N_NEI, 64))
    out_ref[...] = jnp.sum(sm * gf3, axis=1)


def _final_pass(g, pe, featT, acc2, aw1, ab1, ag1, abe1, aw2, ab2, rb):
    rows = g.shape[0]
    pb = rb // N_NEI
    hid = aw1.shape[0]
    cnt = float(rows)
    return pl.pallas_call(
        functools.partial(_final_body, cnt=cnt),
        grid=(rows // rb,),
        in_specs=[
            pl.BlockSpec((rb, D_TAB), lambda i: (i, 0)),
            pl.BlockSpec((rb, 64), lambda i: (i, 0)),
            pl.BlockSpec((pb, 64), lambda i: (i, 0)),
            pl.BlockSpec((72, 64), lambda i: (0, 0)),
            pl.BlockSpec((hid, 64), lambda i: (0, 0)),
            pl.BlockSpec((1, hid), lambda i: (0, 0)),
            pl.BlockSpec((1, hid), lambda i: (0, 0)),
            pl.BlockSpec((1, hid), lambda i: (0, 0)),
            pl.BlockSpec((64, hid), lambda i: (0, 0)),
            pl.BlockSpec((1, 64), lambda i: (0, 0)),
        ],
        out_specs=pl.BlockSpec((pb, 64), lambda i: (i, 0)),
        out_shape=jax.ShapeDtypeStruct((rows // N_NEI, 64), jnp.float32),
    )(g, pe, featT, acc2, aw1, ab1, ag1, abe1, aw2, ab2)


def kernel(pcd, feat, pcd_feadb, feat_feadb,
           pos_w1, pos_b1, pos_g1, pos_be1, pos_w2, pos_b2,
           attn_w1, attn_b1, attn_g1, attn_be1, attn_w2, attn_b2):
    B, C, N = feat.shape
    M = N + feat_feadb.shape[2]
    rows = B * N * N_NEI
    RB = 2048

    fusion_feat = jnp.concatenate([feat, feat_feadb], axis=2)
    fusion_pcd = jnp.concatenate([pcd, pcd_feadb], axis=2)
    rT = fusion_feat.transpose(0, 2, 1)
    pT = fusion_pcd.transpose(0, 2, 1)
    table = jnp.concatenate(
        [rT, pT, jnp.zeros((B, M, D_TAB - C - 3), jnp.float32)],
        axis=2).reshape(B * M, D_TAB)
    qT = feat.transpose(0, 2, 1)
    featT = qT.reshape(B * N, C)
    pcd16 = jnp.concatenate(
        [pcd.transpose(0, 2, 1).reshape(B * N, 3),
         jnp.zeros((B * N, 13), jnp.float32)], axis=1)

    idx = _knn(qT, rT)
    g = _sc_gather(table, idx.reshape(rows))
    acc1 = _stats1(g, pcd16, RB)

    w1p = jnp.concatenate(
        [pos_w1, jnp.zeros((pos_w1.shape[0], 13), jnp.float32)], axis=1)
    pe, acc2 = _pe_pass(g, pcd16, featT, acc1, w1p,
                        pos_b1[None, :], pos_g1[None, :], pos_be1[None, :],
                        pos_w2, pos_b2[None, :], RB)
    refined = _final_pass(g, pe, featT, acc2, attn_w1,
                          attn_b1[None, :], attn_g1[None, :],
                          attn_be1[None, :], attn_w2, attn_b2[None, :], RB)
    return refined.reshape(B, N, C).transpose(0, 2, 1)


# speedup vs baseline: 12.7878x; 1.1280x over previous
import functools

import jax
import jax.numpy as jnp
from jax.experimental import pallas as pl
from jax.experimental.pallas import tpu as pltpu
from jax.experimental.pallas import tpu_sc as plsc

N_NEI = 16
D_TAB = 80
EPS = 1e-5


def _knn_body(q_ref, r_ref, idx_ref, *, m_total):
    b = pl.program_id(0)
    q = q_ref[0]
    r = r_ref[0]
    qs = jnp.sum(q * q, axis=1, keepdims=True)
    rs = jnp.sum(r * r, axis=1)[None, :]
    d = qs + rs - 2.0 * jax.lax.dot_general(
        q, r, (((1,), (1,)), ((), ())), preferred_element_type=jnp.float32)
    nq = d.shape[0]
    nv = m_total // 128
    inf = jnp.float32(jnp.inf)
    d3 = jnp.reshape(d, (nq, nv, 128))
    cmin = jnp.min(d3, axis=1)
    liota = jax.lax.broadcasted_iota(jnp.int32, (nq, 128), 1)
    lsel = []
    for _ in range(N_NEI):
        lj = jnp.argmin(cmin, axis=1)[:, None]
        lsel.append(lj)
        cmin = jnp.where(liota == lj, inf, cmin)
    lanes = jnp.concatenate(lsel, axis=1)
    dparts = []
    gparts = []
    for c in range(nv):
        dparts.append(jnp.take_along_axis(d[:, c * 128:(c + 1) * 128],
                                          lanes, axis=1))
        gparts.append(lanes + c * 128)
    dc = jnp.concatenate(dparts, axis=1)
    gidx = jnp.concatenate(gparts, axis=1)
    big = jnp.int32(m_total)
    cols = []
    for _ in range(N_NEI):
        mv = jnp.min(dc, axis=1, keepdims=True)
        jg = jnp.min(jnp.where(dc == mv, gidx, big), axis=1, keepdims=True)
        cols.append(jg)
        dc = jnp.where(gidx == jg, inf, dc)
    idx_ref[0] = jnp.concatenate(cols, axis=1) + b * m_total


def _knn(qT, rT):
    B, N, C = qT.shape
    M = rT.shape[1]
    NQ = 256
    return pl.pallas_call(
        functools.partial(_knn_body, m_total=M),
        grid=(B, N // NQ),
        in_specs=[
            pl.BlockSpec((1, NQ, C), lambda b, i: (b, i, 0)),
            pl.BlockSpec((1, M, C), lambda b, i: (b, 0, 0)),
        ],
        out_specs=pl.BlockSpec((1, NQ, N_NEI), lambda b, i: (b, i, 0)),
        out_shape=jax.ShapeDtypeStruct((B, N, N_NEI), jnp.int32),
    )(qT, rT)


def _sc_gather(table, idx_flat):
    rows_total = idx_flat.shape[0]
    d = table.shape[1]
    info = plsc.get_sparse_core_info()
    nw = info.num_cores * info.num_subcores
    per_w = rows_total // nw
    ch = 128
    n_ch = per_w // ch
    mesh = plsc.VectorSubcoreMesh(core_axis_name="c", subcore_axis_name="s")

    @functools.partial(
        pl.kernel,
        out_type=jax.ShapeDtypeStruct((rows_total, d), jnp.float32),
        mesh=mesh,
        scratch_types=[
            pltpu.VMEM((ch,), jnp.int32),
            pltpu.VMEM((ch, d), jnp.float32),
            pltpu.SemaphoreType.DMA,
        ],
        compiler_params=pltpu.CompilerParams(use_tc_tiling_on_sc=False),
    )
    def k(table_hbm, idx_hbm, out_hbm, idx_v, rows_v, sem):
        wid = jax.lax.axis_index("s") * info.num_cores + jax.lax.axis_index("c")
        base = wid * per_w

        def body(c, carry):
            off = base + c * ch
            pltpu.sync_copy(idx_hbm.at[pl.ds(off, ch)], idx_v)
            pltpu.async_copy(table_hbm.at[idx_v], rows_v, sem).wait()
            pltpu.sync_copy(rows_v, out_hbm.at[pl.ds(off, ch)])
            return carry

        jax.lax.fori_loop(0, n_ch, body, 0)

    return k(table, idx_flat)


def _stats1_body(gp_ref, p_ref, acc_ref):
    pb = p_ref.shape[0]
    gp = gp_ref[:, 64:80]
    p = p_ref[...]
    prep = jnp.reshape(
        jnp.broadcast_to(p[:, None, :], (pb, N_NEI, 16)), (pb * N_NEI, 16))
    pr = prep - gp
    outer = jax.lax.dot_general(
        pr, pr, (((0,), (0,)), ((), ())), preferred_element_type=jnp.float32)
    s = jnp.sum(pr, axis=0)

    @pl.when(pl.program_id(0) == 0)
    def _():
        acc_ref[...] = jnp.zeros_like(acc_ref)

    acc_ref[0:16, :] += outer
    acc_ref[16:17, :] += s[None, :]


def _stats1(g, pcd16, rb):
    rows = g.shape[0]
    pb = rb // N_NEI
    return pl.pallas_call(
        _stats1_body,
        grid=(rows // rb,),
        in_specs=[
            pl.BlockSpec((rb, D_TAB), lambda i: (i, 0)),
            pl.BlockSpec((pb, 16), lambda i: (i, 0)),
        ],
        out_specs=pl.BlockSpec((24, 16), lambda i: (0, 0)),
        out_shape=jax.ShapeDtypeStruct((24, 16), jnp.float32),
    )(g, pcd16)


def _pe_body(g_ref, p_ref, f_ref, acc1_ref, w1_ref, b1_ref, g1_ref, be1_ref,
             w2_ref, b2_ref, pe_ref, acc2_ref, *, cnt):
    pb = p_ref.shape[0]
    rb = pb * N_NEI
    s = acc1_ref[16:17, :]
    outer = acc1_ref[0:16, :]
    mean_p = s / cnt
    cov = outer / cnt - mean_p * jnp.reshape(mean_p, (16, 1))
    w1 = w1_ref[...]
    mean1 = jax.lax.dot_general(
        mean_p, w1, (((1,), (1,)), ((), ())),
        preferred_element_type=jnp.float32) + b1_ref[...]
    wc = jax.lax.dot_general(
        w1, cov, (((1,), (0,)), ((), ())), preferred_element_type=jnp.float32)
    var1 = jnp.reshape(jnp.sum(wc * w1, axis=1), (1, 64))

    gp = g_ref[:, 64:80]
    p = p_ref[...]
    prep = jnp.reshape(
        jnp.broadcast_to(p[:, None, :], (pb, N_NEI, 16)), (rb, 16))
    pr = prep - gp
    pe1 = jax.lax.dot_general(
        pr, w1, (((1,), (1,)), ((), ())),
        preferred_element_type=jnp.float32) + b1_ref[...]
    xn = (pe1 - mean1) * jax.lax.rsqrt(var1 + EPS) * g1_ref[...] + be1_ref[...]
    z = jnp.maximum(xn, 0.0)
    pe = jax.lax.dot_general(
        z, w2_ref[...], (((1,), (1,)), ((), ())),
        preferred_element_type=jnp.float32) + b2_ref[...]
    pe_ref[...] = pe

    f = f_ref[...]
    frep = jnp.reshape(
        jnp.broadcast_to(f[:, None, :], (pb, N_NEI, 64)), (rb, 64))
    x2 = (frep - g_ref[:, 0:64]) + pe
    outer2 = jax.lax.dot_general(
        x2, x2, (((0,), (0,)), ((), ())), preferred_element_type=jnp.float32)
    s2 = jnp.sum(x2, axis=0)

    @pl.when(pl.program_id(0) == 0)
    def _():
        acc2_ref[...] = jnp.zeros_like(acc2_ref)

    acc2_ref[0:64, :] += outer2
    acc2_ref[64:65, :] += s2[None, :]


def _pe_pass(g, pcd16, featT, acc1, w1p, b1, g1, be1, w2, b2, rb):
    rows = g.shape[0]
    pb = rb // N_NEI
    cnt = float(rows)
    return pl.pallas_call(
        functools.partial(_pe_body, cnt=cnt),
        grid=(rows // rb,),
        in_specs=[
            pl.BlockSpec((rb, D_TAB), lambda i: (i, 0)),
            pl.BlockSpec((pb, 16), lambda i: (i, 0)),
            pl.BlockSpec((pb, 64), lambda i: (i, 0)),
            pl.BlockSpec((24, 16), lambda i: (0, 0)),
            pl.BlockSpec((64, 16), lambda i: (0, 0)),
            pl.BlockSpec((1, 64), lambda i: (0, 0)),
            pl.BlockSpec((1, 64), lambda i: (0, 0)),
            pl.BlockSpec((1, 64), lambda i: (0, 0)),
            pl.BlockSpec((64, 64), lambda i: (0, 0)),
            pl.BlockSpec((1, 64), lambda i: (0, 0)),
        ],
        out_specs=[
            pl.BlockSpec((rb, 64), lambda i: (i, 0)),
            pl.BlockSpec((72, 64), lambda i: (0, 0)),
        ],
        out_shape=[
            jax.ShapeDtypeStruct((rows, 64), jnp.float32),
            jax.ShapeDtypeStruct((72, 64), jnp.float32),
        ],
    )(g, pcd16, featT, acc1, w1p, b1, g1, be1, w2, b2)


def _final_body(g_ref, pe_ref, f_ref, acc2_ref, w1_ref, b1_ref, g1_ref,
                be1_ref, w2_ref, b2_ref, out_ref, *, cnt):
    pb = f_ref.shape[0]
    rb = pb * N_NEI
    hid = w1_ref.shape[0]
    s2 = acc2_ref[64:65, :]
    outer2 = acc2_ref[0:64, :]
    mean_x = s2 / cnt
    cov = outer2 / cnt - mean_x * jnp.reshape(mean_x, (64, 1))
    w1 = w1_ref[...]
    mean2 = jax.lax.dot_general(
        mean_x, w1, (((1,), (1,)), ((), ())),
        preferred_element_type=jnp.float32) + b1_ref[...]
    wc = jax.lax.dot_general(
        w1, cov, (((1,), (0,)), ((), ())), preferred_element_type=jnp.float32)
    var2 = jnp.reshape(jnp.sum(wc * w1, axis=1), (1, hid))

    pe = pe_ref[...]
    f = f_ref[...]
    frep = jnp.reshape(
        jnp.broadcast_to(f[:, None, :], (pb, N_NEI, 64)), (rb, 64))
    gfeat = g_ref[:, 0:64]
    x2 = (frep - gfeat) + pe
    ap = jax.lax.dot_general(
        x2, w1, (((1,), (1,)), ((), ())),
        preferred_element_type=jnp.float32) + b1_ref[...]
    an = (ap - mean2) * jax.lax.rsqrt(var2 + EPS) * g1_ref[...] + be1_ref[...]
    an = jnp.maximum(an, 0.0)
    wp = jax.lax.dot_general(
        an, w2_ref[...], (((1,), (1,)), ((), ())),
        preferred_element_type=jnp.float32) + b2_ref[...]
    wp3 = jnp.reshape(wp, (pb, N_NEI, 64))
    m = jnp.max(wp3, axis=1, keepdims=True)
    e = jnp.exp(wp3 - m)
    sm = e / jnp.sum(e, axis=1, keepdims=True)
    gf3 = jnp.reshape(gfeat + pe, (pb, N_NEI, 64))
    out_ref[...] = jnp.sum(sm * gf3, axis=1)


def _final_pass(g, pe, featT, acc2, aw1, ab1, ag1, abe1, aw2, ab2, rb):
    rows = g.shape[0]
    pb = rb // N_NEI
    hid = aw1.shape[0]
    cnt = float(rows)
    return pl.pallas_call(
        functools.partial(_final_body, cnt=cnt),
        grid=(rows // rb,),
        in_specs=[
            pl.BlockSpec((rb, D_TAB), lambda i: (i, 0)),
            pl.BlockSpec((rb, 64), lambda i: (i, 0)),
            pl.BlockSpec((pb, 64), lambda i: (i, 0)),
            pl.BlockSpec((72, 64), lambda i: (0, 0)),
            pl.BlockSpec((hid, 64), lambda i: (0, 0)),
            pl.BlockSpec((1, hid), lambda i: (0, 0)),
            pl.BlockSpec((1, hid), lambda i: (0, 0)),
            pl.BlockSpec((1, hid), lambda i: (0, 0)),
            pl.BlockSpec((64, hid), lambda i: (0, 0)),
            pl.BlockSpec((1, 64), lambda i: (0, 0)),
        ],
        out_specs=pl.BlockSpec((pb, 64), lambda i: (i, 0)),
        out_shape=jax.ShapeDtypeStruct((rows // N_NEI, 64), jnp.float32),
    )(g, pe, featT, acc2, aw1, ab1, ag1, abe1, aw2, ab2)


def kernel(pcd, feat, pcd_feadb, feat_feadb,
           pos_w1, pos_b1, pos_g1, pos_be1, pos_w2, pos_b2,
           attn_w1, attn_b1, attn_g1, attn_be1, attn_w2, attn_b2):
    B, C, N = feat.shape
    M = N + feat_feadb.shape[2]
    rows = B * N * N_NEI
    RB = 2048

    fusion_feat = jnp.concatenate([feat, feat_feadb], axis=2)
    fusion_pcd = jnp.concatenate([pcd, pcd_feadb], axis=2)
    rT = fusion_feat.transpose(0, 2, 1)
    pT = fusion_pcd.transpose(0, 2, 1)
    table = jnp.concatenate(
        [rT, pT, jnp.zeros((B, M, D_TAB - C - 3), jnp.float32)],
        axis=2).reshape(B * M, D_TAB)
    qT = feat.transpose(0, 2, 1)
    featT = qT.reshape(B * N, C)
    pcd16 = jnp.concatenate(
        [pcd.transpose(0, 2, 1).reshape(B * N, 3),
         jnp.zeros((B * N, 13), jnp.float32)], axis=1)

    idx = _knn(qT, rT)
    g = _sc_gather(table, idx.reshape(rows))
    acc1 = _stats1(g, pcd16, RB)

    w1p = jnp.concatenate(
        [pos_w1, jnp.zeros((pos_w1.shape[0], 13), jnp.float32)], axis=1)
    pe, acc2 = _pe_pass(g, pcd16, featT, acc1, w1p,
                        pos_b1[None, :], pos_g1[None, :], pos_be1[None, :],
                        pos_w2, pos_b2[None, :], RB)
    refined = _final_pass(g, pe, featT, acc2, attn_w1,
                          attn_b1[None, :], attn_g1[None, :],
                          attn_be1[None, :], attn_w2, attn_b2[None, :], RB)
    return refined.reshape(B, N, C).transpose(0, 2, 1)
